# baseline (speedup 1.0000x reference)
import numpy as np
import jax
import jax.numpy as jnp
from jax.experimental import pallas as pl
from jax.experimental.pallas import tpu as pltpu

_H = 8
_DH = 64
_M = 16
_DP = 16
_POS_SCALE = 0.1
_EPS = 1e-6
_NEG = -1e30


def _plane_kernel(feat_ref, gains_ref, a_ref, bl_ref, br_ref, tq_ref):
    feat = feat_ref[0]
    c = feat[:, 0:_M]
    s = feat[:, _M:2 * _M]
    pf = feat[:, 2 * _M:3 * _M]
    one = jnp.ones_like(c)
    zr = jnp.zeros_like(c)
    a_ref[0] = jnp.concatenate([c, c - s * s, one] * _H, axis=1)
    bl_ref[0] = jnp.concatenate([-s, zr, zr] * _H, axis=1)
    br_ref[0] = jnp.concatenate([zr, c * s, zr] * _H, axis=1)
    tq_ref[0] = jnp.concatenate(
        [gains_ref[h:h + 1, :] * pf for h in range(_H)], axis=1)


def _attn_kernel(x_ref, wq_ref, wk_ref, wv_ref, wo_ref,
                 a_ref, bl_ref, br_ref, tq_ref, feat_ref,
                 bias_ref, omask_ref, out_ref):
    f32 = jnp.float32
    xb = x_ref[0, 0]
    r = jax.lax.rsqrt(jnp.mean(xb * xb, axis=-1, keepdims=True) + _EPS)
    xn = xb * r
    _rhs_t = (((1,), (1,)), ((), ()))
    q0 = jax.lax.dot_general(xn, wq_ref[...], _rhs_t, preferred_element_type=f32)
    k0 = jax.lax.dot_general(xn, wk_ref[...], _rhs_t, preferred_element_type=f32)
    v = jax.lax.dot_general(xn, wv_ref[...], _rhs_t, preferred_element_type=f32)

    def roll_l(u):
        return jnp.concatenate([u[:, _M:], u[:, :_M]], axis=1)

    def roll_r(u):
        return jnp.concatenate([u[:, -_M:], u[:, :-_M]], axis=1)

    A = a_ref[0]
    Bl = bl_ref[0]
    Br = br_ref[0]
    q = q0 * A + roll_l(q0) * Bl + roll_r(q0) * Br
    k = k0 * A + roll_l(k0) * Bl + roll_r(k0) * Br
    tqc = tq_ref[0]
    pf = feat_ref[0][:, 2 * _M:3 * _M]

    bias = bias_ref[0, 0]
    scale = 1.0 / np.sqrt(_DH)
    _W3 = 3 * _M
    outs = []
    for h in range(_H):
        sl = slice(h * _DH, (h + 1) * _DH)
        qh = jnp.concatenate([q[:, h * _W3:(h + 1) * _W3],
                              tqc[:, h * _DP:(h + 1) * _DP]], axis=1)
        kh = jnp.concatenate([k[:, h * _W3:(h + 1) * _W3], pf], axis=1)
        s = jax.lax.dot_general(qh, kh, (((1,), (1,)), ((), ())),
                                preferred_element_type=f32) * scale + bias
        m = jnp.max(s, axis=-1, keepdims=True)
        e = jnp.exp(s - m)
        p = e / jnp.sum(e, axis=-1, keepdims=True)
        outs.append(jnp.dot(p, v[:, sl], preferred_element_type=f32))
    o = jnp.concatenate(outs, axis=1)
    y = jax.lax.dot_general(o, wo_ref[...], (((1,), (1,)), ((), ())),
                            preferred_element_type=f32)
    out_ref[0, 0] = y * omask_ref[0, 0]


def kernel(x, point_positions, neuron_pad_mask, spike_mask, Wq, Wk, Wv, Wo,
           rms_w, rope_dirs, rope_freqs, rff_Omega, posC_W, pos_head_gain):
    f32 = jnp.float32
    B, T, N, D = x.shape
    pp = point_positions
    angles = jnp.einsum('bnd,fd->bnf', pp, rope_dirs) * rope_freqs
    th = angles[..., :_M]
    c = jnp.cos(th)
    s = jnp.sin(th)
    proj = jnp.einsum('bnd,md->bnm', pp, rff_Omega)
    phi = jnp.concatenate([jnp.cos(proj), jnp.sin(proj)], axis=-1)
    pos_feat = jnp.einsum('bnm,pm->bnp', phi, posC_W)
    feat = jnp.concatenate([c, s, pos_feat, jnp.zeros_like(c)], axis=-1)
    gains = _POS_SCALE * pos_head_gain
    perm = np.concatenate([
        h * _DH + np.concatenate([np.arange(0, 2 * _M, 2),
                                  np.arange(1, 2 * _M, 2),
                                  np.arange(2 * _M, 3 * _M)])
        for h in range(_H)])
    Wq_p = (Wq * rms_w[None, :])[perm, :]
    Wk_p = (Wk * rms_w[None, :])[perm, :]
    Wv_s = Wv * rms_w[None, :]
    WoT = Wo
    valid = neuron_pad_mask != 0
    spk = (spike_mask != 0) & valid[:, None, :]
    bias = jnp.where(spk, 0.0, _NEG).astype(f32)[:, :, None, :]
    has_key = jnp.any(spk, axis=-1)
    omask = (valid[:, None, :] & has_key[:, :, None]).astype(f32)[..., None]

    planes = pl.pallas_call(
        _plane_kernel,
        grid=(B,),
        in_specs=[
            pl.BlockSpec((1, N, 4 * _M), lambda b: (b, 0, 0)),
            pl.BlockSpec((_H, _DP), lambda b: (0, 0)),
        ],
        out_specs=[pl.BlockSpec((1, N, 3 * _M * _H), lambda b: (b, 0, 0))] * 3
        + [pl.BlockSpec((1, N, _DP * _H), lambda b: (b, 0, 0))],
        out_shape=[jax.ShapeDtypeStruct((B, N, 3 * _M * _H), f32)] * 3
        + [jax.ShapeDtypeStruct((B, N, _DP * _H), f32)],
    )(feat, gains)
    A, Bl, Br, Tq = planes

    out = pl.pallas_call(
        _attn_kernel,
        grid=(B, T),
        in_specs=[
            pl.BlockSpec((1, 1, N, D), lambda b, t: (b, t, 0, 0)),
            pl.BlockSpec((3 * _M * _H, D), lambda b, t: (0, 0)),
            pl.BlockSpec((3 * _M * _H, D), lambda b, t: (0, 0)),
            pl.BlockSpec((D, D), lambda b, t: (0, 0)),
            pl.BlockSpec((D, D), lambda b, t: (0, 0)),
            pl.BlockSpec((1, N, 3 * _M * _H), lambda b, t: (b, 0, 0)),
            pl.BlockSpec((1, N, 3 * _M * _H), lambda b, t: (b, 0, 0)),
            pl.BlockSpec((1, N, 3 * _M * _H), lambda b, t: (b, 0, 0)),
            pl.BlockSpec((1, N, _DP * _H), lambda b, t: (b, 0, 0)),
            pl.BlockSpec((1, N, 4 * _M), lambda b, t: (b, 0, 0)),
            pl.BlockSpec((1, 1, 1, N), lambda b, t: (b, t, 0, 0)),
            pl.BlockSpec((1, 1, N, 1), lambda b, t: (b, t, 0, 0)),
        ],
        out_specs=pl.BlockSpec((1, 1, N, D), lambda b, t: (b, t, 0, 0)),
        out_shape=jax.ShapeDtypeStruct((B, T, N, D), f32),
        compiler_params=pltpu.CompilerParams(
            dimension_semantics=("parallel", "parallel")),
    )(x, Wq_p, Wk_p, Wv_s, WoT, A, Bl, Br, Tq, feat, bias, omask)
    return out


# speedup vs baseline: 1.2130x; 1.2130x over previous
import numpy as np
import jax
import jax.numpy as jnp
from jax.experimental import pallas as pl
from jax.experimental.pallas import tpu as pltpu

_H = 8
_DH = 64
_M = 16
_DP = 16
_POS_SCALE = 0.1
_EPS = 1e-6
_NEG = -1e30


def _plane_kernel(feat_ref, gains_ref, a_ref, bl_ref, br_ref, tq_ref, tk_ref):
    feat = feat_ref[0]
    c = feat[:, 0:_M]
    s = feat[:, _M:2 * _M]
    pf = feat[:, 2 * _M:3 * _M]
    one = jnp.ones_like(c)
    zr = jnp.zeros_like(c)
    a_ref[0] = jnp.concatenate([c, c - s * s, one, zr] * _H, axis=1)
    bl_ref[0] = jnp.concatenate([-s, zr, zr, zr] * _H, axis=1)
    br_ref[0] = jnp.concatenate([zr, c * s, zr, zr] * _H, axis=1)
    tq_parts = []
    for h in range(_H):
        tq_parts.extend([zr, zr, zr, gains_ref[h:h + 1, :] * pf])
    tq_ref[0] = jnp.concatenate(tq_parts, axis=1)
    tk_ref[0] = jnp.concatenate([zr, zr, zr, pf] * _H, axis=1)


def _attn_kernel(x_ref, wq_ref, wk_ref, wv_ref, wo_ref,
                 a_ref, bl_ref, br_ref, tq_ref, tk_ref,
                 bias_ref, omask_ref, out_ref):
    f32 = jnp.float32
    xb = x_ref[0, 0]
    r = jax.lax.rsqrt(jnp.mean(xb * xb, axis=-1, keepdims=True) + _EPS)
    xn = xb * r
    _rhs_t = (((1,), (1,)), ((), ()))
    q0 = jax.lax.dot_general(xn, wq_ref[...], _rhs_t, preferred_element_type=f32)
    k0 = jax.lax.dot_general(xn, wk_ref[...], _rhs_t, preferred_element_type=f32)
    v = jax.lax.dot_general(xn, wv_ref[...], _rhs_t, preferred_element_type=f32)

    def roll_l(u):
        return jnp.concatenate([u[:, _M:], u[:, :_M]], axis=1)

    def roll_r(u):
        return jnp.concatenate([u[:, -_M:], u[:, :-_M]], axis=1)

    A = a_ref[0]
    Bl = bl_ref[0]
    Br = br_ref[0]
    q = q0 * A + roll_l(q0) * Bl + roll_r(q0) * Br + tq_ref[0]
    k = k0 * A + roll_l(k0) * Bl + roll_r(k0) * Br + tk_ref[0]

    bias = bias_ref[0, 0]
    scale = 1.0 / np.sqrt(_DH)
    outs = []
    for h in range(_H):
        sl = slice(h * _DH, (h + 1) * _DH)
        qh = q[:, sl]
        kh = k[:, sl]
        s = jax.lax.dot_general(qh, kh, (((1,), (1,)), ((), ())),
                                preferred_element_type=f32) * scale + bias
        m = jnp.max(s, axis=-1, keepdims=True)
        e = jnp.exp(s - m)
        rs = 1.0 / jnp.sum(e, axis=-1, keepdims=True)
        oh = jnp.dot(e, v[:, sl], preferred_element_type=f32)
        outs.append(oh * rs)
    o = jnp.concatenate(outs, axis=1)
    y = jax.lax.dot_general(o, wo_ref[...], (((1,), (1,)), ((), ())),
                            preferred_element_type=f32)
    out_ref[0, 0] = y * omask_ref[0, 0]


def kernel(x, point_positions, neuron_pad_mask, spike_mask, Wq, Wk, Wv, Wo,
           rms_w, rope_dirs, rope_freqs, rff_Omega, posC_W, pos_head_gain):
    f32 = jnp.float32
    B, T, N, D = x.shape
    pp = point_positions
    angles = jnp.einsum('bnd,fd->bnf', pp, rope_dirs) * rope_freqs
    th = angles[..., :_M]
    c = jnp.cos(th)
    s = jnp.sin(th)
    proj = jnp.einsum('bnd,md->bnm', pp, rff_Omega)
    phi = jnp.concatenate([jnp.cos(proj), jnp.sin(proj)], axis=-1)
    pos_feat = jnp.einsum('bnm,pm->bnp', phi, posC_W)
    feat = jnp.concatenate([c, s, pos_feat, jnp.zeros_like(c)], axis=-1)
    gains = _POS_SCALE * pos_head_gain
    perm = np.concatenate([
        h * _DH + np.concatenate([np.arange(0, 2 * _M, 2),
                                  np.arange(1, 2 * _M, 2),
                                  np.arange(2 * _M, _DH)])
        for h in range(_H)])
    Wq_p = (Wq * rms_w[None, :])[perm, :]
    Wk_p = (Wk * rms_w[None, :])[perm, :]
    Wv_s = Wv * rms_w[None, :]
    WoT = Wo
    valid = neuron_pad_mask != 0
    spk = (spike_mask != 0) & valid[:, None, :]
    bias = jnp.where(spk, 0.0, _NEG).astype(f32)[:, :, None, :]
    has_key = jnp.any(spk, axis=-1)
    omask = (valid[:, None, :] & has_key[:, :, None]).astype(f32)[..., None]

    planes = pl.pallas_call(
        _plane_kernel,
        grid=(B,),
        in_specs=[
            pl.BlockSpec((1, N, 4 * _M), lambda b: (b, 0, 0)),
            pl.BlockSpec((_H, _DP), lambda b: (0, 0)),
        ],
        out_specs=[pl.BlockSpec((1, N, D), lambda b: (b, 0, 0))] * 5,
        out_shape=[jax.ShapeDtypeStruct((B, N, D), f32)] * 5,
    )(feat, gains)
    A, Bl, Br, Tq, Tk = planes

    out = pl.pallas_call(
        _attn_kernel,
        grid=(B, T),
        in_specs=[
            pl.BlockSpec((1, 1, N, D), lambda b, t: (b, t, 0, 0)),
            pl.BlockSpec((D, D), lambda b, t: (0, 0)),
            pl.BlockSpec((D, D), lambda b, t: (0, 0)),
            pl.BlockSpec((D, D), lambda b, t: (0, 0)),
            pl.BlockSpec((D, D), lambda b, t: (0, 0)),
            pl.BlockSpec((1, N, D), lambda b, t: (b, 0, 0)),
            pl.BlockSpec((1, N, D), lambda b, t: (b, 0, 0)),
            pl.BlockSpec((1, N, D), lambda b, t: (b, 0, 0)),
            pl.BlockSpec((1, N, D), lambda b, t: (b, 0, 0)),
            pl.BlockSpec((1, N, D), lambda b, t: (b, 0, 0)),
            pl.BlockSpec((1, 1, 1, N), lambda b, t: (b, t, 0, 0)),
            pl.BlockSpec((1, 1, N, 1), lambda b, t: (b, t, 0, 0)),
        ],
        out_specs=pl.BlockSpec((1, 1, N, D), lambda b, t: (b, t, 0, 0)),
        out_shape=jax.ShapeDtypeStruct((B, T, N, D), f32),
        compiler_params=pltpu.CompilerParams(
            dimension_semantics=("parallel", "parallel")),
    )(x, Wq_p, Wk_p, Wv_s, WoT, A, Bl, Br, Tq, Tk, bias, omask)
    return out
